# baseline (speedup 1.0000x reference)
import functools

import jax
import jax.numpy as jnp
from jax import lax
from jax.experimental import pallas as pl
from jax.experimental.pallas import tpu as pltpu
from jax.experimental.pallas import tpu_sc as plsc

N = 10000
E = 320000
D_IN = 128
H1 = 64
H2 = 32
D_OUT = 128

NC = 2
NS = 16
NW = NC * NS
NP = 10240
NPT = NP // NS
EPAD = NW * 10240 - E
EP = (E + EPAD) // NW
CHUNK = 128
NCHUNK = EP // CHUNK

W1 = H1 + 16
W2 = H2


def _make_sc_scatter(W):
  mesh = plsc.VectorSubcoreMesh(core_axis_name="c", subcore_axis_name="s")

  @functools.partial(
      pl.kernel,
      out_type=jax.ShapeDtypeStruct((NC, NP, 128), jnp.float32),
      mesh=mesh,
      compiler_params=pltpu.CompilerParams(use_tc_tiling_on_sc=False),
      scratch_types=[
          pltpu.VMEM((EP,), jnp.int32),
          pltpu.VMEM((EP,), jnp.int32),
          [pltpu.VMEM((CHUNK, W), jnp.float32) for _ in range(4)],
          pltpu.VMEM_SHARED((NP, W), jnp.float32),
          [pltpu.SemaphoreType.DMA for _ in range(4)],
          [pltpu.SemaphoreType.DMA for _ in range(4)],
          pltpu.SemaphoreType.DMA,
      ],
  )
  def sc_kernel(t_hbm, src_hbm, dst_hbm, zero_hbm, out_hbm,
                src_v, dst_v, rows, acc_s, gsem, ssem, semp):
    c = lax.axis_index("c")
    s = lax.axis_index("s")
    w = c * NS + s
    z = pltpu.async_copy(zero_hbm, acc_s.at[pl.ds(s * NPT, NPT)], semp)
    a = pltpu.async_copy(src_hbm.at[pl.ds(w * EP, EP)], src_v, semp)
    b = pltpu.async_copy(dst_hbm.at[pl.ds(w * EP, EP)], dst_v, semp)
    z.wait()
    a.wait()
    b.wait()
    plsc.subcore_barrier()

    def gwait(b_):
      pltpu.make_async_copy(t_hbm.at[src_v.at[pl.ds(0, CHUNK)]], rows[b_],
                            gsem[b_]).wait()

    def swait(b_):
      pltpu.make_async_copy(rows[b_], acc_s.at[dst_v.at[pl.ds(0, CHUNK)]],
                            ssem[b_]).wait()

    for k in range(3):
      pltpu.async_copy(t_hbm.at[src_v.at[pl.ds(k * CHUNK, CHUNK)]], rows[k],
                       gsem[k])

    def body(jj, carry):
      j = 4 * jj
      for k in range(4):
        ck = j + k
        bp = (k + 3) % 4
        gwait(k)
        pltpu.async_copy(rows[k], acc_s.at[dst_v.at[pl.ds(ck * CHUNK, CHUNK)]],
                         ssem[k], add=True)

        @pl.when(ck + 3 < NCHUNK)
        def _(ck=ck, bp=bp, k=k):
          if k == 0:
            @pl.when(jj >= 1)
            def _():
              swait(bp)
          else:
            swait(bp)
          pltpu.async_copy(
              t_hbm.at[src_v.at[pl.ds((ck + 3) * CHUNK, CHUNK)]],
              rows[bp], gsem[bp])

      return carry

    lax.fori_loop(0, NCHUNK // 4, body, 0)
    for k in range(4):
      swait(k)
    plsc.subcore_barrier()
    pltpu.sync_copy(acc_s.at[pl.ds(s * NPT, NPT)],
                    out_hbm.at[c, pl.ds(s * NPT, NPT), pl.ds(0, W)])

  return sc_kernel


_R = 10000


def _tc_pre(x, Wl1, Wr1):
  def body(x_ref, wl_ref, wr_ref, taug_ref, r1_ref):
    xb = x_ref[...]
    t = jnp.dot(xb, wl_ref[...], preferred_element_type=jnp.float32)
    taug_ref[...] = jnp.concatenate(
        [t, jnp.ones((_R, 1), jnp.float32),
         jnp.zeros((_R, W1 - H1 - 1), jnp.float32)], axis=1)
    r1_ref[...] = jnp.dot(xb, wr_ref[...], preferred_element_type=jnp.float32)

  return pl.pallas_call(
      body,
      grid=(N // _R,),
      in_specs=[pl.BlockSpec((_R, D_IN), lambda i: (i, 0)),
                pl.BlockSpec((D_IN, H1), lambda i: (0, 0)),
                pl.BlockSpec((D_IN, H1), lambda i: (0, 0))],
      out_specs=[pl.BlockSpec((_R, W1), lambda i: (i, 0)),
                 pl.BlockSpec((_R, H1), lambda i: (i, 0))],
      out_shape=[jax.ShapeDtypeStruct((N, W1), jnp.float32),
                 jax.ShapeDtypeStruct((N, H1), jnp.float32)],
  )(x, Wl1, Wr1)


def _tc_mid(parts1, r1, bl1, Wl2, Wr2):
  def body(p_ref, r1_ref, bl1_ref, wl2_ref, wr2_ref, t2_ref, r2_ref, cnt_ref):
    s1 = p_ref[0, :, :W1] + p_ref[1, :, :W1]
    cnt = jnp.maximum(s1[:, H1:H1 + 1], 1.0)
    mean = s1[:, :H1] / cnt
    h1 = jnp.maximum(mean + bl1_ref[...][None, :] + r1_ref[...], 0.0)
    t2_ref[...] = jnp.dot(h1, wl2_ref[...], preferred_element_type=jnp.float32)
    r2_ref[...] = jnp.dot(h1, wr2_ref[...], preferred_element_type=jnp.float32)
    cnt_ref[...] = jnp.broadcast_to(cnt, (_R, 8))

  return pl.pallas_call(
      body,
      grid=(N // _R,),
      in_specs=[pl.BlockSpec((NC, _R, 128), lambda i: (0, i, 0)),
                pl.BlockSpec((_R, H1), lambda i: (i, 0)),
                pl.BlockSpec((H1,), lambda i: (0,)),
                pl.BlockSpec((H1, H2), lambda i: (0, 0)),
                pl.BlockSpec((H1, H2), lambda i: (0, 0))],
      out_specs=[pl.BlockSpec((_R, H2), lambda i: (i, 0)),
                 pl.BlockSpec((_R, H2), lambda i: (i, 0)),
                 pl.BlockSpec((_R, 8), lambda i: (i, 0))],
      out_shape=[jax.ShapeDtypeStruct((N, H2), jnp.float32),
                 jax.ShapeDtypeStruct((N, H2), jnp.float32),
                 jax.ShapeDtypeStruct((N, 8), jnp.float32)],
  )(parts1, r1, bl1, Wl2, Wr2)


def _tc_post(parts2, cnt8, r2, bl2, Wfc, bfc):
  def body(p_ref, cnt_ref, r2_ref, bl2_ref, wfc_ref, bfc_ref, emb_ref, out_ref):
    s2 = p_ref[0, :, :H2] + p_ref[1, :, :H2]
    mean2 = s2 / cnt_ref[:, 0:1]
    h2 = jnp.maximum(mean2 + bl2_ref[...][None, :] + r2_ref[...], 0.0)
    emb_ref[...] = h2
    out_ref[...] = (jnp.dot(h2, wfc_ref[...], preferred_element_type=jnp.float32)
                    + bfc_ref[...][None, :])

  return pl.pallas_call(
      body,
      grid=(N // _R,),
      in_specs=[pl.BlockSpec((NC, _R, 128), lambda i: (0, i, 0)),
                pl.BlockSpec((_R, 8), lambda i: (i, 0)),
                pl.BlockSpec((_R, H2), lambda i: (i, 0)),
                pl.BlockSpec((H2,), lambda i: (0,)),
                pl.BlockSpec((H2, D_OUT), lambda i: (0, 0)),
                pl.BlockSpec((D_OUT,), lambda i: (0,))],
      out_specs=[pl.BlockSpec((_R, H2), lambda i: (i, 0)),
                 pl.BlockSpec((_R, D_OUT), lambda i: (i, 0))],
      out_shape=[jax.ShapeDtypeStruct((N, H2), jnp.float32),
                 jax.ShapeDtypeStruct((N, D_OUT), jnp.float32)],
  )(parts2, cnt8, r2, bl2, Wfc, bfc)


_sc_scatter_w1 = _make_sc_scatter(W1)
_sc_scatter_w2 = _make_sc_scatter(W2)


def kernel(x, edge_index, Wl1, bl1, Wr1, Wl2, bl2, Wr2, Wfc, bfc):
  src1d = jnp.concatenate([edge_index[0], jnp.zeros((EPAD,), jnp.int32)])
  dst1d = jnp.concatenate(
      [edge_index[1], jnp.full((EPAD,), NP - 1, jnp.int32)])
  zeros1 = jnp.zeros((NPT, W1), jnp.float32)
  zeros2 = jnp.zeros((NPT, W2), jnp.float32)

  t1aug, r1 = _tc_pre(x, Wl1, Wr1)
  parts1 = _sc_scatter_w1(t1aug, src1d, dst1d, zeros1)
  t2, r2, cnt8 = _tc_mid(parts1, r1, bl1, Wl2, Wr2)
  parts2 = _sc_scatter_w2(t2, src1d, dst1d, zeros2)
  embedding, out = _tc_post(parts2, cnt8, r2, bl2, Wfc, bfc)
  return (embedding, out)


# speedup vs baseline: 1.0176x; 1.0176x over previous
import functools

import jax
import jax.numpy as jnp
from jax import lax
from jax.experimental import pallas as pl
from jax.experimental.pallas import tpu as pltpu
from jax.experimental.pallas import tpu_sc as plsc

N = 10000
E = 320000
D_IN = 128
H1 = 64
H2 = 32
D_OUT = 128

NC = 2
NS = 16
NW = NC * NS
NP = 10240
NPT = NP // NS
EPAD = NW * 10240 - E
EP = (E + EPAD) // NW
CHUNK = 128
NCHUNK = EP // CHUNK

W1 = H1 + 16
W2 = H2


def _make_sc_scatter(W):
  mesh = plsc.VectorSubcoreMesh(core_axis_name="c", subcore_axis_name="s")

  @functools.partial(
      pl.kernel,
      out_type=jax.ShapeDtypeStruct((NC, NP, 128), jnp.float32),
      mesh=mesh,
      compiler_params=pltpu.CompilerParams(use_tc_tiling_on_sc=False),
      scratch_types=[
          pltpu.VMEM((EP,), jnp.int32),
          pltpu.VMEM((EP,), jnp.int32),
          [pltpu.VMEM((CHUNK, W), jnp.float32) for _ in range(4)],
          pltpu.VMEM_SHARED((NP, W), jnp.float32),
          [pltpu.SemaphoreType.DMA for _ in range(4)],
          [pltpu.SemaphoreType.DMA for _ in range(4)],
          pltpu.SemaphoreType.DMA,
      ],
  )
  def sc_kernel(t_hbm, src_hbm, dst_hbm, zero_hbm, out_hbm,
                src_v, dst_v, rows, acc_s, gsem, ssem, semp):
    c = lax.axis_index("c")
    s = lax.axis_index("s")
    w = c * NS + s
    z = pltpu.async_copy(zero_hbm, acc_s.at[pl.ds(s * NPT, NPT)], semp)
    a = pltpu.async_copy(src_hbm.at[pl.ds(w * EP, EP)], src_v, semp)
    b = pltpu.async_copy(dst_hbm.at[pl.ds(w * EP, EP)], dst_v, semp)
    z.wait()
    a.wait()
    b.wait()
    plsc.subcore_barrier()

    def gwait(b_):
      pltpu.make_async_copy(t_hbm.at[src_v.at[pl.ds(0, CHUNK)]], rows[b_],
                            gsem[b_]).wait()

    def swait(b_):
      pltpu.make_async_copy(rows[b_], acc_s.at[dst_v.at[pl.ds(0, CHUNK)]],
                            ssem[b_]).wait()

    for k in range(3):
      pltpu.async_copy(t_hbm.at[src_v.at[pl.ds(k * CHUNK, CHUNK)]], rows[k],
                       gsem[k])

    def body(jj, carry):
      j = 4 * jj
      for k in range(4):
        ck = j + k
        bp = (k + 3) % 4
        gwait(k)
        pltpu.async_copy(rows[k], acc_s.at[dst_v.at[pl.ds(ck * CHUNK, CHUNK)]],
                         ssem[k], add=True)

        @pl.when(ck + 3 < NCHUNK)
        def _(ck=ck, bp=bp, k=k):
          if k == 0:
            @pl.when(jj >= 1)
            def _():
              swait(bp)
          else:
            swait(bp)
          pltpu.async_copy(
              t_hbm.at[src_v.at[pl.ds((ck + 3) * CHUNK, CHUNK)]],
              rows[bp], gsem[bp])

      return carry

    lax.fori_loop(0, NCHUNK // 4, body, 0)
    for k in range(4):
      swait(k)
    plsc.subcore_barrier()
    pltpu.sync_copy(acc_s.at[pl.ds(s * NPT, NPT)],
                    out_hbm.at[c, pl.ds(s * NPT, NPT), pl.ds(0, W)])

  return sc_kernel


_R = 10000


def _tc_pre(x, Wl1, Wr1):
  def body(x_ref, wl_ref, wr_ref, taug_ref, r1_ref):
    xb = x_ref[...]
    t = jnp.dot(xb, wl_ref[...], preferred_element_type=jnp.float32)
    taug_ref[...] = jnp.concatenate(
        [t, jnp.ones((_R, 1), jnp.float32),
         jnp.zeros((_R, W1 - H1 - 1), jnp.float32)], axis=1)
    r1_ref[...] = jnp.dot(xb, wr_ref[...], preferred_element_type=jnp.float32)

  return pl.pallas_call(
      body,
      grid=(N // _R,),
      in_specs=[pl.BlockSpec((_R, D_IN), lambda i: (i, 0)),
                pl.BlockSpec((D_IN, H1), lambda i: (0, 0)),
                pl.BlockSpec((D_IN, H1), lambda i: (0, 0))],
      out_specs=[pl.BlockSpec((_R, W1), lambda i: (i, 0)),
                 pl.BlockSpec((_R, H1), lambda i: (i, 0))],
      out_shape=[jax.ShapeDtypeStruct((N, W1), jnp.float32),
                 jax.ShapeDtypeStruct((N, H1), jnp.float32)],
  )(x, Wl1, Wr1)


def _tc_mid(parts1, r1, bl1, Wl2, Wr2):
  def body(p_ref, r1_ref, bl1_ref, wl2_ref, wr2_ref, t2_ref, r2_ref, cnt_ref):
    s1 = p_ref[0, :, :W1] + p_ref[1, :, :W1]
    cnt = jnp.maximum(s1[:, H1:H1 + 1], 1.0)
    mean = s1[:, :H1] / cnt
    h1 = jnp.maximum(mean + bl1_ref[...][None, :] + r1_ref[...], 0.0)
    t2_ref[...] = jnp.dot(h1, wl2_ref[...], preferred_element_type=jnp.float32)
    r2_ref[...] = jnp.dot(h1, wr2_ref[...], preferred_element_type=jnp.float32)
    cnt_ref[...] = jnp.broadcast_to(cnt, (_R, 8))

  return pl.pallas_call(
      body,
      grid=(N // _R,),
      in_specs=[pl.BlockSpec((NC, _R, 128), lambda i: (0, i, 0)),
                pl.BlockSpec((_R, H1), lambda i: (i, 0)),
                pl.BlockSpec((H1,), lambda i: (0,)),
                pl.BlockSpec((H1, H2), lambda i: (0, 0)),
                pl.BlockSpec((H1, H2), lambda i: (0, 0))],
      out_specs=[pl.BlockSpec((_R, H2), lambda i: (i, 0)),
                 pl.BlockSpec((_R, H2), lambda i: (i, 0)),
                 pl.BlockSpec((_R, 8), lambda i: (i, 0))],
      out_shape=[jax.ShapeDtypeStruct((N, H2), jnp.float32),
                 jax.ShapeDtypeStruct((N, H2), jnp.float32),
                 jax.ShapeDtypeStruct((N, 8), jnp.float32)],
  )(parts1, r1, bl1, Wl2, Wr2)


def _tc_post(parts2, cnt8, r2, bl2, Wfc, bfc):
  def body(p_ref, cnt_ref, r2_ref, bl2_ref, wfc_ref, bfc_ref, emb_ref, out_ref):
    s2 = p_ref[0, :, :H2] + p_ref[1, :, :H2]
    mean2 = s2 / cnt_ref[:, 0:1]
    h2 = jnp.maximum(mean2 + bl2_ref[...][None, :] + r2_ref[...], 0.0)
    emb_ref[...] = h2
    out_ref[...] = (jnp.dot(h2, wfc_ref[...], preferred_element_type=jnp.float32)
                    + bfc_ref[...][None, :])

  return pl.pallas_call(
      body,
      grid=(N // _R,),
      in_specs=[pl.BlockSpec((NC, _R, 128), lambda i: (0, i, 0)),
                pl.BlockSpec((_R, 8), lambda i: (i, 0)),
                pl.BlockSpec((_R, H2), lambda i: (i, 0)),
                pl.BlockSpec((H2,), lambda i: (0,)),
                pl.BlockSpec((H2, D_OUT), lambda i: (0, 0)),
                pl.BlockSpec((D_OUT,), lambda i: (0,))],
      out_specs=[pl.BlockSpec((_R, H2), lambda i: (i, 0)),
                 pl.BlockSpec((_R, D_OUT), lambda i: (i, 0))],
      out_shape=[jax.ShapeDtypeStruct((N, H2), jnp.float32),
                 jax.ShapeDtypeStruct((N, D_OUT), jnp.float32)],
  )(parts2, cnt8, r2, bl2, Wfc, bfc)


_sc_scatter_w1 = _make_sc_scatter(W1)
_sc_scatter_w2 = _make_sc_scatter(W2)


def kernel(x, edge_index, Wl1, bl1, Wr1, Wl2, bl2, Wr2, Wfc, bfc):
  src1d = jnp.concatenate([edge_index[0], jnp.zeros((EPAD,), jnp.int32)])
  dst1d = jnp.concatenate(
      [edge_index[1], N + (jnp.arange(EPAD, dtype=jnp.int32) % (NP - N))])
  zeros1 = jnp.zeros((NPT, W1), jnp.float32)
  zeros2 = jnp.zeros((NPT, W2), jnp.float32)

  t1aug, r1 = _tc_pre(x, Wl1, Wr1)
  parts1 = _sc_scatter_w1(t1aug, src1d, dst1d, zeros1)
  t2, r2, cnt8 = _tc_mid(parts1, r1, bl1, Wl2, Wr2)
  parts2 = _sc_scatter_w2(t2, src1d, dst1d, zeros2)
  embedding, out = _tc_post(parts2, cnt8, r2, bl2, Wfc, bfc)
  return (embedding, out)


# speedup vs baseline: 2.9306x; 2.8799x over previous
import functools

import jax
import jax.numpy as jnp
from jax import lax
from jax.experimental import pallas as pl
from jax.experimental.pallas import tpu as pltpu
from jax.experimental.pallas import tpu_sc as plsc

N = 10000
E = 320000
D_IN = 128
H1 = 64
H2 = 32
D_OUT = 128

NC = 2
NS = 16
NW = NC * NS
NP = 10240
NPT = NP // NS
EPAD = NW * 10240 - E
EP = (E + EPAD) // NW
CHUNK = 128
NCHUNK = EP // CHUNK

W1 = H1 + 16
W2 = H2


def _make_sc_scatter(W):
  mesh = plsc.VectorSubcoreMesh(core_axis_name="c", subcore_axis_name="s")

  @functools.partial(
      pl.kernel,
      out_type=jax.ShapeDtypeStruct((NC, NP, 128), jnp.float32),
      mesh=mesh,
      compiler_params=pltpu.CompilerParams(use_tc_tiling_on_sc=False),
      scratch_types=[
          pltpu.VMEM((EP,), jnp.int32),
          pltpu.VMEM((EP,), jnp.int32),
          [pltpu.VMEM((CHUNK, W), jnp.float32) for _ in range(4)],
          pltpu.VMEM_SHARED((NP, W), jnp.float32),
          [pltpu.SemaphoreType.DMA for _ in range(4)],
          [pltpu.SemaphoreType.DMA for _ in range(4)],
          pltpu.SemaphoreType.DMA,
      ],
  )
  def sc_kernel(t_hbm, src_hbm, dst_hbm, zero_hbm, out_hbm,
                src_v, dst_v, rows, acc_s, gsem, ssem, semp):
    c = lax.axis_index("c")
    s = lax.axis_index("s")
    w = c * NS + s
    z = pltpu.async_copy(zero_hbm, acc_s.at[pl.ds(s * NPT, NPT)], semp)
    a = pltpu.async_copy(src_hbm.at[pl.ds(w * EP, EP)], src_v, semp)
    b = pltpu.async_copy(dst_hbm.at[pl.ds(w * EP, EP)], dst_v, semp)
    z.wait()
    a.wait()
    b.wait()
    plsc.subcore_barrier()

    def gwait(b_):
      pltpu.make_async_copy(t_hbm.at[src_v.at[pl.ds(0, CHUNK)]], rows[b_],
                            gsem[b_]).wait()

    def swait(b_):
      pltpu.make_async_copy(rows[b_], acc_s.at[dst_v.at[pl.ds(0, CHUNK)]],
                            ssem[b_]).wait()

    for k in range(3):
      pltpu.async_copy(t_hbm.at[src_v.at[pl.ds(k * CHUNK, CHUNK)]], rows[k],
                       gsem[k])

    def body(jj, carry):
      j = 4 * jj
      for k in range(4):
        ck = j + k
        bp = (k + 3) % 4
        gwait(k)
        pltpu.async_copy(rows[k], acc_s.at[dst_v.at[pl.ds(ck * CHUNK, CHUNK)]],
                         ssem[k], add=True)

        @pl.when(ck + 3 < NCHUNK)
        def _(ck=ck, bp=bp, k=k):
          if k == 0:
            @pl.when(jj >= 1)
            def _():
              swait(bp)
          else:
            swait(bp)
          pltpu.async_copy(
              t_hbm.at[src_v.at[pl.ds((ck + 3) * CHUNK, CHUNK)]],
              rows[bp], gsem[bp])

      return carry

    lax.fori_loop(0, NCHUNK // 4, body, 0)
    for k in range(4):
      swait(k)
    plsc.subcore_barrier()
    pltpu.sync_copy(acc_s.at[pl.ds(s * NPT, NPT)],
                    out_hbm.at[c, pl.ds(s * NPT, NPT), pl.ds(0, W)])

  return sc_kernel


_R = 10000


def _tc_pre(x, Wl1, Wr1):
  def body(x_ref, wl_ref, wr_ref, taug_ref, r1_ref):
    xb = x_ref[...]
    t = jnp.dot(xb, wl_ref[...], preferred_element_type=jnp.float32)
    taug_ref[...] = jnp.concatenate(
        [t, jnp.ones((_R, 1), jnp.float32),
         jnp.zeros((_R, W1 - H1 - 1), jnp.float32)], axis=1)
    r1_ref[...] = jnp.dot(xb, wr_ref[...], preferred_element_type=jnp.float32)

  return pl.pallas_call(
      body,
      grid=(N // _R,),
      in_specs=[pl.BlockSpec((_R, D_IN), lambda i: (i, 0)),
                pl.BlockSpec((D_IN, H1), lambda i: (0, 0)),
                pl.BlockSpec((D_IN, H1), lambda i: (0, 0))],
      out_specs=[pl.BlockSpec((_R, W1), lambda i: (i, 0)),
                 pl.BlockSpec((_R, H1), lambda i: (i, 0))],
      out_shape=[jax.ShapeDtypeStruct((N, W1), jnp.float32),
                 jax.ShapeDtypeStruct((N, H1), jnp.float32)],
  )(x, Wl1, Wr1)


def _tc_mid(parts1, r1, bl1, Wl2, Wr2):
  def body(p_ref, r1_ref, bl1_ref, wl2_ref, wr2_ref, t2_ref, r2_ref, cnt_ref):
    s1 = p_ref[0, :, :W1] + p_ref[1, :, :W1]
    cnt = jnp.maximum(s1[:, H1:H1 + 1], 1.0)
    mean = s1[:, :H1] / cnt
    h1 = jnp.maximum(mean + bl1_ref[...][None, :] + r1_ref[...], 0.0)
    t2_ref[...] = jnp.dot(h1, wl2_ref[...], preferred_element_type=jnp.float32)
    r2_ref[...] = jnp.dot(h1, wr2_ref[...], preferred_element_type=jnp.float32)
    cnt_ref[...] = jnp.broadcast_to(cnt, (_R, 8))

  return pl.pallas_call(
      body,
      grid=(N // _R,),
      in_specs=[pl.BlockSpec((NC, _R, 128), lambda i: (0, i, 0)),
                pl.BlockSpec((_R, H1), lambda i: (i, 0)),
                pl.BlockSpec((H1,), lambda i: (0,)),
                pl.BlockSpec((H1, H2), lambda i: (0, 0)),
                pl.BlockSpec((H1, H2), lambda i: (0, 0))],
      out_specs=[pl.BlockSpec((_R, H2), lambda i: (i, 0)),
                 pl.BlockSpec((_R, H2), lambda i: (i, 0)),
                 pl.BlockSpec((_R, 8), lambda i: (i, 0))],
      out_shape=[jax.ShapeDtypeStruct((N, H2), jnp.float32),
                 jax.ShapeDtypeStruct((N, H2), jnp.float32),
                 jax.ShapeDtypeStruct((N, 8), jnp.float32)],
  )(parts1, r1, bl1, Wl2, Wr2)


def _tc_post(parts2, cnt8, r2, bl2, Wfc, bfc):
  def body(p_ref, cnt_ref, r2_ref, bl2_ref, wfc_ref, bfc_ref, emb_ref, out_ref):
    s2 = p_ref[0, :, :H2] + p_ref[1, :, :H2]
    mean2 = s2 / cnt_ref[:, 0:1]
    h2 = jnp.maximum(mean2 + bl2_ref[...][None, :] + r2_ref[...], 0.0)
    emb_ref[...] = h2
    out_ref[...] = (jnp.dot(h2, wfc_ref[...], preferred_element_type=jnp.float32)
                    + bfc_ref[...][None, :])

  return pl.pallas_call(
      body,
      grid=(N // _R,),
      in_specs=[pl.BlockSpec((NC, _R, 128), lambda i: (0, i, 0)),
                pl.BlockSpec((_R, 8), lambda i: (i, 0)),
                pl.BlockSpec((_R, H2), lambda i: (i, 0)),
                pl.BlockSpec((H2,), lambda i: (0,)),
                pl.BlockSpec((H2, D_OUT), lambda i: (0, 0)),
                pl.BlockSpec((D_OUT,), lambda i: (0,))],
      out_specs=[pl.BlockSpec((_R, H2), lambda i: (i, 0)),
                 pl.BlockSpec((_R, D_OUT), lambda i: (i, 0))],
      out_shape=[jax.ShapeDtypeStruct((N, H2), jnp.float32),
                 jax.ShapeDtypeStruct((N, D_OUT), jnp.float32)],
  )(parts2, cnt8, r2, bl2, Wfc, bfc)


_sc_scatter_w1 = _make_sc_scatter(W1)
_sc_scatter_w2 = _make_sc_scatter(W2)


def kernel(x, edge_index, Wl1, bl1, Wr1, Wl2, bl2, Wr2, Wfc, bfc):
  src1d = jnp.concatenate(
      [edge_index[0], jnp.arange(EPAD, dtype=jnp.int32) % N])
  dst1d = jnp.concatenate(
      [edge_index[1], N + (jnp.arange(EPAD, dtype=jnp.int32) % (NP - N))])
  zeros1 = jnp.zeros((NPT, W1), jnp.float32)
  zeros2 = jnp.zeros((NPT, W2), jnp.float32)

  t1aug, r1 = _tc_pre(x, Wl1, Wr1)
  parts1 = _sc_scatter_w1(t1aug, src1d, dst1d, zeros1)
  t2, r2, cnt8 = _tc_mid(parts1, r1, bl1, Wl2, Wr2)
  parts2 = _sc_scatter_w2(t2, src1d, dst1d, zeros2)
  embedding, out = _tc_post(parts2, cnt8, r2, bl2, Wfc, bfc)
  return (embedding, out)


# speedup vs baseline: 3.0187x; 1.0301x over previous
import functools

import jax
import jax.numpy as jnp
from jax import lax
from jax.experimental import pallas as pl
from jax.experimental.pallas import tpu as pltpu
from jax.experimental.pallas import tpu_sc as plsc

N = 10000
E = 320000
D_IN = 128
H1 = 64
H2 = 32
D_OUT = 128

NC = 2
NS = 16
NW = NC * NS
NP = 10240
NPT = NP // NS
EPAD = NW * 10240 - E
EP = (E + EPAD) // NW
CHUNK = 128
NCHUNK = EP // CHUNK
NB = 5

W1 = H1 + 16
W2 = H2


def _make_sc_scatter(W):
  mesh = plsc.VectorSubcoreMesh(core_axis_name="c", subcore_axis_name="s")

  @functools.partial(
      pl.kernel,
      out_type=jax.ShapeDtypeStruct((NC, NP, 128), jnp.float32),
      mesh=mesh,
      compiler_params=pltpu.CompilerParams(use_tc_tiling_on_sc=False),
      scratch_types=[
          pltpu.VMEM((EP,), jnp.int32),
          pltpu.VMEM((EP,), jnp.int32),
          [pltpu.VMEM((CHUNK, W), jnp.float32) for _ in range(NB)],
          pltpu.VMEM_SHARED((NP, W), jnp.float32),
          [pltpu.SemaphoreType.DMA for _ in range(NB)],
          [pltpu.SemaphoreType.DMA for _ in range(NB)],
          pltpu.SemaphoreType.DMA,
      ],
  )
  def sc_kernel(t_hbm, src_hbm, dst_hbm, zero_hbm, out_hbm,
                src_v, dst_v, rows, acc_s, gsem, ssem, semp):
    c = lax.axis_index("c")
    s = lax.axis_index("s")
    w = c * NS + s
    z = pltpu.async_copy(zero_hbm, acc_s.at[pl.ds(s * NPT, NPT)], semp)
    a = pltpu.async_copy(src_hbm.at[pl.ds(w * EP, EP)], src_v, semp)
    b = pltpu.async_copy(dst_hbm.at[pl.ds(w * EP, EP)], dst_v, semp)
    z.wait()
    a.wait()
    b.wait()
    plsc.subcore_barrier()

    def gwait(b_):
      pltpu.make_async_copy(t_hbm.at[src_v.at[pl.ds(0, CHUNK)]], rows[b_],
                            gsem[b_]).wait()

    def swait(b_):
      pltpu.make_async_copy(rows[b_], acc_s.at[dst_v.at[pl.ds(0, CHUNK)]],
                            ssem[b_]).wait()

    for k in range(NB - 1):
      pltpu.async_copy(t_hbm.at[src_v.at[pl.ds(k * CHUNK, CHUNK)]], rows[k],
                       gsem[k])

    def body(jj, carry):
      j = NB * jj
      for k in range(NB):
        ck = j + k
        bp = (k + NB - 1) % NB
        gwait(k)
        pltpu.async_copy(rows[k], acc_s.at[dst_v.at[pl.ds(ck * CHUNK, CHUNK)]],
                         ssem[k], add=True)

        @pl.when(ck + NB - 1 < NCHUNK)
        def _(ck=ck, bp=bp, k=k):
          if k == 0:
            @pl.when(jj >= 1)
            def _():
              swait(bp)
          else:
            swait(bp)
          pltpu.async_copy(
              t_hbm.at[src_v.at[pl.ds((ck + NB - 1) * CHUNK, CHUNK)]],
              rows[bp], gsem[bp])

      return carry

    lax.fori_loop(0, NCHUNK // NB, body, 0)
    for k in range(NB):
      swait(k)
    plsc.subcore_barrier()
    pltpu.sync_copy(acc_s.at[pl.ds(s * NPT, NPT)],
                    out_hbm.at[c, pl.ds(s * NPT, NPT), pl.ds(0, W)])

  return sc_kernel


_R = 10000


def _tc_pre(x, Wl1, Wr1):
  def body(x_ref, wl_ref, wr_ref, taug_ref, r1_ref):
    xb = x_ref[...]
    t = jnp.dot(xb, wl_ref[...], preferred_element_type=jnp.float32)
    taug_ref[...] = jnp.concatenate(
        [t, jnp.ones((_R, 1), jnp.float32),
         jnp.zeros((_R, W1 - H1 - 1), jnp.float32)], axis=1)
    r1_ref[...] = jnp.dot(xb, wr_ref[...], preferred_element_type=jnp.float32)

  return pl.pallas_call(
      body,
      grid=(N // _R,),
      in_specs=[pl.BlockSpec((_R, D_IN), lambda i: (i, 0)),
                pl.BlockSpec((D_IN, H1), lambda i: (0, 0)),
                pl.BlockSpec((D_IN, H1), lambda i: (0, 0))],
      out_specs=[pl.BlockSpec((_R, W1), lambda i: (i, 0)),
                 pl.BlockSpec((_R, H1), lambda i: (i, 0))],
      out_shape=[jax.ShapeDtypeStruct((N, W1), jnp.float32),
                 jax.ShapeDtypeStruct((N, H1), jnp.float32)],
  )(x, Wl1, Wr1)


def _tc_mid(parts1, r1, bl1, Wl2, Wr2):
  def body(p_ref, r1_ref, bl1_ref, wl2_ref, wr2_ref, t2_ref, r2_ref, cnt_ref):
    s1 = p_ref[0, :, :W1] + p_ref[1, :, :W1]
    cnt = jnp.maximum(s1[:, H1:H1 + 1], 1.0)
    mean = s1[:, :H1] / cnt
    h1 = jnp.maximum(mean + bl1_ref[...][None, :] + r1_ref[...], 0.0)
    t2_ref[...] = jnp.dot(h1, wl2_ref[...], preferred_element_type=jnp.float32)
    r2_ref[...] = jnp.dot(h1, wr2_ref[...], preferred_element_type=jnp.float32)
    cnt_ref[...] = jnp.broadcast_to(cnt, (_R, 8))

  return pl.pallas_call(
      body,
      grid=(N // _R,),
      in_specs=[pl.BlockSpec((NC, _R, 128), lambda i: (0, i, 0)),
                pl.BlockSpec((_R, H1), lambda i: (i, 0)),
                pl.BlockSpec((H1,), lambda i: (0,)),
                pl.BlockSpec((H1, H2), lambda i: (0, 0)),
                pl.BlockSpec((H1, H2), lambda i: (0, 0))],
      out_specs=[pl.BlockSpec((_R, H2), lambda i: (i, 0)),
                 pl.BlockSpec((_R, H2), lambda i: (i, 0)),
                 pl.BlockSpec((_R, 8), lambda i: (i, 0))],
      out_shape=[jax.ShapeDtypeStruct((N, H2), jnp.float32),
                 jax.ShapeDtypeStruct((N, H2), jnp.float32),
                 jax.ShapeDtypeStruct((N, 8), jnp.float32)],
  )(parts1, r1, bl1, Wl2, Wr2)


def _tc_post(parts2, cnt8, r2, bl2, Wfc, bfc):
  def body(p_ref, cnt_ref, r2_ref, bl2_ref, wfc_ref, bfc_ref, emb_ref, out_ref):
    s2 = p_ref[0, :, :H2] + p_ref[1, :, :H2]
    mean2 = s2 / cnt_ref[:, 0:1]
    h2 = jnp.maximum(mean2 + bl2_ref[...][None, :] + r2_ref[...], 0.0)
    emb_ref[...] = h2
    out_ref[...] = (jnp.dot(h2, wfc_ref[...], preferred_element_type=jnp.float32)
                    + bfc_ref[...][None, :])

  return pl.pallas_call(
      body,
      grid=(N // _R,),
      in_specs=[pl.BlockSpec((NC, _R, 128), lambda i: (0, i, 0)),
                pl.BlockSpec((_R, 8), lambda i: (i, 0)),
                pl.BlockSpec((_R, H2), lambda i: (i, 0)),
                pl.BlockSpec((H2,), lambda i: (0,)),
                pl.BlockSpec((H2, D_OUT), lambda i: (0, 0)),
                pl.BlockSpec((D_OUT,), lambda i: (0,))],
      out_specs=[pl.BlockSpec((_R, H2), lambda i: (i, 0)),
                 pl.BlockSpec((_R, D_OUT), lambda i: (i, 0))],
      out_shape=[jax.ShapeDtypeStruct((N, H2), jnp.float32),
                 jax.ShapeDtypeStruct((N, D_OUT), jnp.float32)],
  )(parts2, cnt8, r2, bl2, Wfc, bfc)


_sc_scatter_w1 = _make_sc_scatter(W1)
_sc_scatter_w2 = _make_sc_scatter(W2)


def kernel(x, edge_index, Wl1, bl1, Wr1, Wl2, bl2, Wr2, Wfc, bfc):
  src1d = jnp.concatenate(
      [edge_index[0], jnp.arange(EPAD, dtype=jnp.int32) % N])
  dst1d = jnp.concatenate(
      [edge_index[1], N + (jnp.arange(EPAD, dtype=jnp.int32) % (NP - N))])
  zeros1 = jnp.zeros((NPT, W1), jnp.float32)
  zeros2 = jnp.zeros((NPT, W2), jnp.float32)

  t1aug, r1 = _tc_pre(x, Wl1, Wr1)
  parts1 = _sc_scatter_w1(t1aug, src1d, dst1d, zeros1)
  t2, r2, cnt8 = _tc_mid(parts1, r1, bl1, Wl2, Wr2)
  parts2 = _sc_scatter_w2(t2, src1d, dst1d, zeros2)
  embedding, out = _tc_post(parts2, cnt8, r2, bl2, Wfc, bfc)
  return (embedding, out)


# speedup vs baseline: 3.1893x; 1.0565x over previous
import functools

import jax
import jax.numpy as jnp
from jax import lax
from jax.experimental import pallas as pl
from jax.experimental.pallas import tpu as pltpu
from jax.experimental.pallas import tpu_sc as plsc

N = 10000
E = 320000
D_IN = 128
H1 = 64
H2 = 32
D_OUT = 128

NC = 2
NS = 16
NW = NC * NS
NP = 10240
NPT = NP // NS
EPAD = NW * 10240 - E
EP = (E + EPAD) // NW
EPR = E // NW
PPT = EPAD // NW
CHUNK = 128
NCHUNK = EP // CHUNK
NB = 5

W1 = H1 + 16
W2 = H2


def _make_sc_scatter(W):
  mesh = plsc.VectorSubcoreMesh(core_axis_name="c", subcore_axis_name="s")

  @functools.partial(
      pl.kernel,
      out_type=jax.ShapeDtypeStruct((NC, NP, 128), jnp.float32),
      mesh=mesh,
      compiler_params=pltpu.CompilerParams(use_tc_tiling_on_sc=False),
      scratch_types=[
          pltpu.VMEM((EP,), jnp.int32),
          pltpu.VMEM((EP,), jnp.int32),
          [pltpu.VMEM((CHUNK, W), jnp.float32) for _ in range(NB)],
          pltpu.VMEM_SHARED((NP, W), jnp.float32),
          [pltpu.SemaphoreType.DMA for _ in range(NB)],
          [pltpu.SemaphoreType.DMA for _ in range(NB)],
          pltpu.SemaphoreType.DMA,
      ],
  )
  def sc_kernel(t_hbm, eflat_hbm, spad_hbm, dpad_hbm, zero_hbm, out_hbm,
                src_v, dst_v, rows, acc_s, gsem, ssem, semp):
    c = lax.axis_index("c")
    s = lax.axis_index("s")
    w = c * NS + s
    z = pltpu.async_copy(zero_hbm, acc_s.at[pl.ds(s * NPT, NPT)], semp)
    a = pltpu.async_copy(eflat_hbm.at[pl.ds(w * EPR, EPR)],
                         src_v.at[pl.ds(0, EPR)], semp)
    b = pltpu.async_copy(eflat_hbm.at[pl.ds(E + w * EPR, EPR)],
                         dst_v.at[pl.ds(0, EPR)], semp)
    ap = pltpu.async_copy(spad_hbm.at[pl.ds(w * PPT, PPT)],
                          src_v.at[pl.ds(EPR, PPT)], semp)
    bp = pltpu.async_copy(dpad_hbm.at[pl.ds(w * PPT, PPT)],
                          dst_v.at[pl.ds(EPR, PPT)], semp)
    z.wait()
    a.wait()
    b.wait()
    ap.wait()
    bp.wait()
    plsc.subcore_barrier()

    def gwait(b_):
      pltpu.make_async_copy(t_hbm.at[src_v.at[pl.ds(0, CHUNK)]], rows[b_],
                            gsem[b_]).wait()

    def swait(b_):
      pltpu.make_async_copy(rows[b_], acc_s.at[dst_v.at[pl.ds(0, CHUNK)]],
                            ssem[b_]).wait()

    for k in range(NB - 1):
      pltpu.async_copy(t_hbm.at[src_v.at[pl.ds(k * CHUNK, CHUNK)]], rows[k],
                       gsem[k])

    def body(jj, carry):
      j = NB * jj
      for k in range(NB):
        ck = j + k
        bp = (k + NB - 1) % NB
        gwait(k)
        pltpu.async_copy(rows[k], acc_s.at[dst_v.at[pl.ds(ck * CHUNK, CHUNK)]],
                         ssem[k], add=True)

        @pl.when(ck + NB - 1 < NCHUNK)
        def _(ck=ck, bp=bp, k=k):
          if k == 0:
            @pl.when(jj >= 1)
            def _():
              swait(bp)
          else:
            swait(bp)
          pltpu.async_copy(
              t_hbm.at[src_v.at[pl.ds((ck + NB - 1) * CHUNK, CHUNK)]],
              rows[bp], gsem[bp])

      return carry

    lax.fori_loop(0, NCHUNK // NB, body, 0)
    for k in range(NB):
      swait(k)
    plsc.subcore_barrier()
    pltpu.sync_copy(acc_s.at[pl.ds(s * NPT, NPT)],
                    out_hbm.at[c, pl.ds(s * NPT, NPT), pl.ds(0, W)])

  return sc_kernel


_R = 10000


def _tc_pre(x, Wl1, Wr1):
  def body(x_ref, wl_ref, wr_ref, taug_ref, r1_ref):
    xb = x_ref[...]
    t = jnp.dot(xb, wl_ref[...], preferred_element_type=jnp.float32)
    taug_ref[...] = jnp.concatenate(
        [t, jnp.ones((_R, 1), jnp.float32),
         jnp.zeros((_R, W1 - H1 - 1), jnp.float32)], axis=1)
    r1_ref[...] = jnp.dot(xb, wr_ref[...], preferred_element_type=jnp.float32)

  return pl.pallas_call(
      body,
      grid=(N // _R,),
      in_specs=[pl.BlockSpec((_R, D_IN), lambda i: (i, 0)),
                pl.BlockSpec((D_IN, H1), lambda i: (0, 0)),
                pl.BlockSpec((D_IN, H1), lambda i: (0, 0))],
      out_specs=[pl.BlockSpec((_R, W1), lambda i: (i, 0)),
                 pl.BlockSpec((_R, H1), lambda i: (i, 0))],
      out_shape=[jax.ShapeDtypeStruct((N, W1), jnp.float32),
                 jax.ShapeDtypeStruct((N, H1), jnp.float32)],
  )(x, Wl1, Wr1)


def _tc_mid(parts1, r1, bl1, Wl2, Wr2):
  def body(p_ref, r1_ref, bl1_ref, wl2_ref, wr2_ref, t2_ref, r2_ref, cnt_ref):
    s1 = p_ref[0, :, :W1] + p_ref[1, :, :W1]
    cnt = jnp.maximum(s1[:, H1:H1 + 1], 1.0)
    mean = s1[:, :H1] / cnt
    h1 = jnp.maximum(mean + bl1_ref[...][None, :] + r1_ref[...], 0.0)
    t2_ref[...] = jnp.dot(h1, wl2_ref[...], preferred_element_type=jnp.float32)
    r2_ref[...] = jnp.dot(h1, wr2_ref[...], preferred_element_type=jnp.float32)
    cnt_ref[...] = jnp.broadcast_to(cnt, (_R, 8))

  return pl.pallas_call(
      body,
      grid=(N // _R,),
      in_specs=[pl.BlockSpec((NC, _R, 128), lambda i: (0, i, 0)),
                pl.BlockSpec((_R, H1), lambda i: (i, 0)),
                pl.BlockSpec((H1,), lambda i: (0,)),
                pl.BlockSpec((H1, H2), lambda i: (0, 0)),
                pl.BlockSpec((H1, H2), lambda i: (0, 0))],
      out_specs=[pl.BlockSpec((_R, H2), lambda i: (i, 0)),
                 pl.BlockSpec((_R, H2), lambda i: (i, 0)),
                 pl.BlockSpec((_R, 8), lambda i: (i, 0))],
      out_shape=[jax.ShapeDtypeStruct((N, H2), jnp.float32),
                 jax.ShapeDtypeStruct((N, H2), jnp.float32),
                 jax.ShapeDtypeStruct((N, 8), jnp.float32)],
  )(parts1, r1, bl1, Wl2, Wr2)


def _tc_post(parts2, cnt8, r2, bl2, Wfc, bfc):
  def body(p_ref, cnt_ref, r2_ref, bl2_ref, wfc_ref, bfc_ref, emb_ref, out_ref):
    s2 = p_ref[0, :, :H2] + p_ref[1, :, :H2]
    mean2 = s2 / cnt_ref[:, 0:1]
    h2 = jnp.maximum(mean2 + bl2_ref[...][None, :] + r2_ref[...], 0.0)
    emb_ref[...] = h2
    out_ref[...] = (jnp.dot(h2, wfc_ref[...], preferred_element_type=jnp.float32)
                    + bfc_ref[...][None, :])

  return pl.pallas_call(
      body,
      grid=(N // _R,),
      in_specs=[pl.BlockSpec((NC, _R, 128), lambda i: (0, i, 0)),
                pl.BlockSpec((_R, 8), lambda i: (i, 0)),
                pl.BlockSpec((_R, H2), lambda i: (i, 0)),
                pl.BlockSpec((H2,), lambda i: (0,)),
                pl.BlockSpec((H2, D_OUT), lambda i: (0, 0)),
                pl.BlockSpec((D_OUT,), lambda i: (0,))],
      out_specs=[pl.BlockSpec((_R, H2), lambda i: (i, 0)),
                 pl.BlockSpec((_R, D_OUT), lambda i: (i, 0))],
      out_shape=[jax.ShapeDtypeStruct((N, H2), jnp.float32),
                 jax.ShapeDtypeStruct((N, D_OUT), jnp.float32)],
  )(parts2, cnt8, r2, bl2, Wfc, bfc)


_sc_scatter_w1 = _make_sc_scatter(W1)
_sc_scatter_w2 = _make_sc_scatter(W2)


def kernel(x, edge_index, Wl1, bl1, Wr1, Wl2, bl2, Wr2, Wfc, bfc):
  eflat = jnp.reshape(edge_index, (2 * E,))
  spad = jnp.arange(EPAD, dtype=jnp.int32) % N
  dpad = N + (jnp.arange(EPAD, dtype=jnp.int32) % (NP - N))
  zeros1 = jnp.zeros((NPT, W1), jnp.float32)
  zeros2 = jnp.zeros((NPT, W2), jnp.float32)

  t1aug, r1 = _tc_pre(x, Wl1, Wr1)
  parts1 = _sc_scatter_w1(t1aug, eflat, spad, dpad, zeros1)
  t2, r2, cnt8 = _tc_mid(parts1, r1, bl1, Wl2, Wr2)
  parts2 = _sc_scatter_w2(t2, eflat, spad, dpad, zeros2)
  embedding, out = _tc_post(parts2, cnt8, r2, bl2, Wfc, bfc)
  return (embedding, out)
